# baseline (speedup 1.0000x reference)
import functools

import jax
import jax.numpy as jnp
from jax import lax
from jax.experimental import pallas as pl
from jax.experimental.pallas import tpu as pltpu
from jax.experimental.pallas import tpu_sc as plsc

_NC = 2
_NS = 16
_CW = 32
_BB = 128
_NBI = 8
_SB = _BB * _NBI


def _round_up(a, b):
    return (a + b - 1) // b * b


@functools.lru_cache(maxsize=None)
def _make_sc_segsum(N, E_pad, NCH, NKEY, NKEYP, with_counts):
    ZPT = NKEYP // _NS
    EPT = E_pad // _NS
    EPC = E_pad // (_NS * _NC)
    CPC = NCH // _NC
    nsb_data = EPT // _SB
    nsb_cnt = EPC // _SB
    EROWS = E_pad // _BB
    f32 = jnp.float32

    mesh = plsc.VectorSubcoreMesh(core_axis_name="c", subcore_axis_name="s")

    def body(xflat, src8, key2, zeros_h, ones_h, *refs):
        if with_counts:
            s_out, c_out, acc, kbuf, ibuf, dbuf, obuf, sem = refs
        else:
            s_out, acc, kbuf, ibuf, dbuf, obuf, sem = refs
        cid = lax.axis_index("c")
        sid = lax.axis_index("s")
        pltpu.sync_copy(ones_h, obuf)

        def zero_acc():
            pltpu.sync_copy(zeros_h, acc.at[pl.ds(sid * ZPT, ZPT)])

        def run_pass(row_base, nsb, chunk):
            def super_step(sb, carry):
                rk = row_base + sb * _NBI
                pltpu.sync_copy(key2.at[pl.ds(rk, _NBI)], kbuf)
                if chunk is not None:
                    ri = chunk * EROWS + rk
                    pltpu.sync_copy(src8.at[pl.ds(ri, _NBI)], ibuf)
                    cps = [pltpu.async_copy(xflat.at[ibuf.at[j]], dbuf.at[j], sem)
                           for j in range(_NBI)]
                    for cp in cps:
                        cp.wait()
                    for j in range(_NBI):
                        pltpu.sync_copy(dbuf.at[j], acc.at[kbuf.at[j]], add=True)
                else:
                    for j in range(_NBI):
                        pltpu.sync_copy(obuf, acc.at[kbuf.at[j]], add=True)
                return carry
            lax.fori_loop(0, nsb, super_step, 0)

        for p in range(CPC):
            chunk = cid * CPC + p
            zero_acc()
            plsc.subcore_barrier()
            run_pass(sid * (EPT // _BB), nsb_data, chunk)
            plsc.subcore_barrier()
            pltpu.sync_copy(acc.at[pl.ds(sid * ZPT, ZPT)],
                            s_out.at[pl.ds(chunk * NKEYP + sid * ZPT, ZPT)])
            plsc.subcore_barrier()

        if with_counts:
            zero_acc()
            plsc.subcore_barrier()
            run_pass(cid * (EPC * _NS // _BB) + sid * (EPC // _BB), nsb_cnt, None)
            plsc.subcore_barrier()
            pltpu.sync_copy(acc.at[pl.ds(sid * ZPT, ZPT)],
                            c_out.at[pl.ds(cid * NKEYP + sid * ZPT, ZPT)])

    out_type = [jax.ShapeDtypeStruct((NCH * NKEYP, _CW), f32)]
    if with_counts:
        out_type.append(jax.ShapeDtypeStruct((_NC * NKEYP, _CW), f32))

    return pl.kernel(
        body,
        out_type=out_type,
        mesh=mesh,
        compiler_params=pltpu.CompilerParams(use_tc_tiling_on_sc=False),
        scratch_types=[
            pltpu.VMEM_SHARED((NKEYP, _CW), f32),
            pltpu.VMEM((_NBI, _BB), jnp.int32),
            pltpu.VMEM((_NBI, _BB), jnp.int32),
            pltpu.VMEM((_NBI, _BB, _CW), f32),
            pltpu.VMEM((_BB, _CW), f32),
            pltpu.SemaphoreType.DMA,
        ],
    )


def _tc_combine(s3, c3, x, WrT_l, WtT_l, b_l, T, N, last):
    NCH, NKEYP, CW = s3.shape
    D = x.shape[1]
    BN = 1000
    assert N % BN == 0
    NJ = N // BN

    def body(s_ref, c_ref, x_ref, wr_ref, wt_ref, b_ref, o_ref):
        i = pl.program_id(1)
        c = pl.program_id(2)

        @pl.when((i == 0) & (c == 0))
        def _root():
            o_ref[...] = jnp.dot(x_ref[...], wt_ref[...],
                                 preferred_element_type=jnp.float32) + b_ref[...]

        @pl.when(i > 0)
        def _rel():
            cs = c_ref[...]
            cv = cs[0, :, 0:1] + cs[1, :, 0:1]
            inv = 1.0 / jnp.maximum(cv, 1.0)
            o_ref[...] += jnp.dot(s_ref[0] * inv, wr_ref[0],
                                  preferred_element_type=jnp.float32)

        @pl.when((i == T) & (c == NCH - 1))
        def _act():
            acc = o_ref[...]
            if last:
                m = jnp.max(acc, axis=-1, keepdims=True)
                ex = jnp.exp(acc - m)
                o_ref[...] = acc - m - jnp.log(jnp.sum(ex, axis=-1,
                                                       keepdims=True))
            else:
                o_ref[...] = jnp.maximum(acc, 0.0)

    def ti(i):
        return jnp.maximum(i - 1, 0)

    return pl.pallas_call(
        body,
        grid=(NJ, T + 1, NCH),
        in_specs=[
            pl.BlockSpec((1, BN, CW), lambda j, i, c: (c, ti(i) * NJ + j, 0)),
            pl.BlockSpec((_NC, BN, CW), lambda j, i, c: (0, ti(i) * NJ + j, 0)),
            pl.BlockSpec((BN, D), lambda j, i, c: (j, 0)),
            pl.BlockSpec((1, CW, D), lambda j, i, c: (ti(i), c, 0)),
            pl.BlockSpec((D, D), lambda j, i, c: (0, 0)),
            pl.BlockSpec((1, D), lambda j, i, c: (0, 0)),
        ],
        out_specs=pl.BlockSpec((BN, D), lambda j, i, c: (j, 0)),
        out_shape=jax.ShapeDtypeStruct((N, D), jnp.float32),
    )(s3, c3, x, WrT_l, WtT_l, b_l.reshape(1, D))


def kernel(x_dict, edge_index, edge_type, node_type, local_node_idx,
           W_rel, W_root, b_root):
    N, D = x_dict.shape
    E = edge_index.shape[1]
    L, T = W_rel.shape[0], W_rel.shape[1]
    NCH = D // _CW
    NKEY = T * N
    NKEYP = _round_up(NKEY + 1, _NS * 8)
    E_pad = _round_up(E, _NS * _NC * _SB)

    src = edge_index[0]
    dst = edge_index[1]
    pad = E_pad - E
    key = edge_type * N + dst
    keyp = jnp.concatenate([key, jnp.full((pad,), NKEY, jnp.int32)])
    srcp = jnp.concatenate([src, jnp.zeros((pad,), jnp.int32)])
    key2 = keyp.reshape(E_pad // _BB, _BB)
    src8 = (srcp[None, :] + (jnp.arange(NCH, dtype=jnp.int32) * N)[:, None])
    src8 = src8.reshape(NCH * E_pad // _BB, _BB)
    zeros_h = jnp.zeros((NKEYP // _NS, _CW), jnp.float32)
    ones_h = jnp.ones((_BB, _CW), jnp.float32)

    WrT = W_rel.transpose(0, 1, 3, 2)
    WtT = W_root[:, 0].transpose(0, 2, 1)
    b = b_root[:, 0]

    def chunkify(h):
        return h.reshape(N, NCH, _CW).transpose(1, 0, 2).reshape(NCH * N, _CW)

    h = x_dict
    cnt = None
    for l in range(L):
        sc = _make_sc_segsum(N, E_pad, NCH, NKEY, NKEYP, l == 0)
        outs = sc(chunkify(h), src8, key2, zeros_h, ones_h)
        if l == 0:
            s_flat, c_flat = outs
            cnt = c_flat.reshape(_NC, NKEYP, _CW)
        else:
            (s_flat,) = outs
        s3 = s_flat.reshape(NCH, NKEYP, _CW)
        h = _tc_combine(s3, cnt, h, WrT[l], WtT[l], b[l], T, N,
                        last=(l == L - 1))
    return h


# speedup vs baseline: 1.5749x; 1.5749x over previous
import functools

import jax
import jax.numpy as jnp
from jax import lax
from jax.experimental import pallas as pl
from jax.experimental.pallas import tpu as pltpu
from jax.experimental.pallas import tpu_sc as plsc

_NC = 2
_NS = 16
_CW = 32
_BB = 128
_NBI = 8
_SB = _BB * _NBI


def _round_up(a, b):
    return (a + b - 1) // b * b


@functools.lru_cache(maxsize=None)
def _make_sc_segsum(N, E_pad, NCH, NKEY, NKEYP, with_counts):
    ZPT = NKEYP // _NS
    EPT = E_pad // _NS
    EPC = E_pad // (_NS * _NC)
    CPC = NCH // _NC
    nsb_data = EPT // _SB
    nsb_cnt = EPC // _SB
    EROWS = E_pad // _BB
    f32 = jnp.float32

    mesh = plsc.VectorSubcoreMesh(core_axis_name="c", subcore_axis_name="s")

    def body(xflat, src8, key2, zeros_h, ones_h, *refs):
        if with_counts:
            s_out, c_out, acc, kbuf, ibuf, dbuf, obuf, sem = refs
        else:
            s_out, acc, kbuf, ibuf, dbuf, obuf, sem = refs
        cid = lax.axis_index("c")
        sid = lax.axis_index("s")
        pltpu.sync_copy(ones_h, obuf)

        def zero_acc():
            pltpu.sync_copy(zeros_h, acc.at[pl.ds(sid * ZPT, ZPT)])

        def run_pass(row_base, nsb, chunk):
            def super_step(sb, carry):
                rk = row_base + sb * _NBI
                pltpu.sync_copy(key2.at[pl.ds(rk, _NBI)], kbuf)
                if chunk is not None:
                    ri = chunk * EROWS + rk
                    pltpu.sync_copy(src8.at[pl.ds(ri, _NBI)], ibuf)
                    cps = [pltpu.async_copy(xflat.at[ibuf.at[j]], dbuf.at[j], sem)
                           for j in range(_NBI)]
                    for cp in cps:
                        cp.wait()
                    for j in range(_NBI):
                        pltpu.sync_copy(dbuf.at[j], acc.at[kbuf.at[j]], add=True)
                else:
                    for j in range(_NBI):
                        pltpu.sync_copy(obuf, acc.at[kbuf.at[j]], add=True)
                return carry
            lax.fori_loop(0, nsb, super_step, 0)

        for p in range(CPC):
            chunk = cid * CPC + p
            zero_acc()
            plsc.subcore_barrier()
            run_pass(sid * (EPT // _BB), nsb_data, chunk)
            plsc.subcore_barrier()
            pltpu.sync_copy(acc.at[pl.ds(sid * ZPT, ZPT)],
                            s_out.at[cid, pl.ds(sid * ZPT, ZPT),
                                     pl.ds(p * _CW, _CW)])
            plsc.subcore_barrier()

        if with_counts:
            zero_acc()
            plsc.subcore_barrier()
            run_pass(cid * (EPC * _NS // _BB) + sid * (EPC // _BB), nsb_cnt, None)
            plsc.subcore_barrier()
            pltpu.sync_copy(acc.at[pl.ds(sid * ZPT, ZPT)],
                            c_out.at[cid, pl.ds(sid * ZPT, ZPT),
                                     pl.ds(0, _CW)])

    out_type = [jax.ShapeDtypeStruct((_NC, NKEYP, CPC * _CW), f32)]
    if with_counts:
        out_type.append(jax.ShapeDtypeStruct((_NC, NKEYP, CPC * _CW), f32))

    return pl.kernel(
        body,
        out_type=out_type,
        mesh=mesh,
        compiler_params=pltpu.CompilerParams(use_tc_tiling_on_sc=False),
        scratch_types=[
            pltpu.VMEM_SHARED((NKEYP, _CW), f32),
            pltpu.VMEM((_NBI, _BB), jnp.int32),
            pltpu.VMEM((_NBI, _BB), jnp.int32),
            pltpu.VMEM((_NBI, _BB, _CW), f32),
            pltpu.VMEM((_BB, _CW), f32),
            pltpu.SemaphoreType.DMA,
        ],
    )


def _tc_combine(s128, c128, x, WrT_l, WtT_l, b_l, T, N, last):
    NC, NKEYP, KW = s128.shape
    D = x.shape[1]
    BN = 1000
    assert N % BN == 0
    NJ = N // BN

    def body(s_ref, c_ref, x_ref, wr_ref, wt_ref, b_ref, o_ref):
        i = pl.program_id(1)

        @pl.when(i == 0)
        def _root():
            o_ref[...] = jnp.dot(x_ref[...], wt_ref[...],
                                 preferred_element_type=jnp.float32) + b_ref[...]

        @pl.when(i > 0)
        def _rel():
            cs = c_ref[...]
            cv = cs[0, :, 0:1] + cs[1, :, 0:1]
            inv = 1.0 / jnp.maximum(cv, 1.0)
            acc = o_ref[...]
            for e in range(NC):
                acc = acc + jnp.dot(s_ref[e] * inv,
                                    wr_ref[0][e * KW:(e + 1) * KW, :],
                                    preferred_element_type=jnp.float32)
            o_ref[...] = acc

        @pl.when(i == T)
        def _act():
            acc = o_ref[...]
            if last:
                m = jnp.max(acc, axis=-1, keepdims=True)
                ex = jnp.exp(acc - m)
                o_ref[...] = acc - m - jnp.log(jnp.sum(ex, axis=-1,
                                                       keepdims=True))
            else:
                o_ref[...] = jnp.maximum(acc, 0.0)

    def ti(i):
        return jnp.maximum(i - 1, 0)

    return pl.pallas_call(
        body,
        grid=(NJ, T + 1),
        in_specs=[
            pl.BlockSpec((NC, BN, KW), lambda j, i: (0, ti(i) * NJ + j, 0)),
            pl.BlockSpec((NC, BN, KW), lambda j, i: (0, ti(i) * NJ + j, 0)),
            pl.BlockSpec((BN, D), lambda j, i: (j, 0)),
            pl.BlockSpec((1, D, D), lambda j, i: (ti(i), 0, 0)),
            pl.BlockSpec((D, D), lambda j, i: (0, 0)),
            pl.BlockSpec((1, D), lambda j, i: (0, 0)),
        ],
        out_specs=pl.BlockSpec((BN, D), lambda j, i: (j, 0)),
        out_shape=jax.ShapeDtypeStruct((N, D), jnp.float32),
    )(s128, c128, x, WrT_l, WtT_l, b_l.reshape(1, D))


def kernel(x_dict, edge_index, edge_type, node_type, local_node_idx,
           W_rel, W_root, b_root):
    N, D = x_dict.shape
    E = edge_index.shape[1]
    L, T = W_rel.shape[0], W_rel.shape[1]
    NCH = D // _CW
    NKEY = T * N
    NKEYP = _round_up(NKEY + 1, _NS * 8)
    E_pad = _round_up(E, _NS * _NC * _SB)

    src = edge_index[0]
    dst = edge_index[1]
    pad = E_pad - E
    key = edge_type * N + dst
    keyp = jnp.concatenate([key, jnp.full((pad,), NKEY, jnp.int32)])
    srcp = jnp.concatenate([src, jnp.zeros((pad,), jnp.int32)])
    key2 = keyp.reshape(E_pad // _BB, _BB)
    src8 = (srcp[None, :] + (jnp.arange(NCH, dtype=jnp.int32) * N)[:, None])
    src8 = src8.reshape(NCH * E_pad // _BB, _BB)
    zeros_h = jnp.zeros((NKEYP // _NS, _CW), jnp.float32)
    ones_h = jnp.ones((_BB, _CW), jnp.float32)

    WrT = W_rel.transpose(0, 1, 3, 2)
    WtT = W_root[:, 0].transpose(0, 2, 1)
    b = b_root[:, 0]

    def chunkify(h):
        return h.reshape(N, NCH, _CW).transpose(1, 0, 2).reshape(NCH * N, _CW)

    h = x_dict
    cnt = None
    for l in range(L):
        sc = _make_sc_segsum(N, E_pad, NCH, NKEY, NKEYP, l == 0)
        outs = sc(chunkify(h), src8, key2, zeros_h, ones_h)
        if l == 0:
            s128, cnt = outs
        else:
            (s128,) = outs
        h = _tc_combine(s128, cnt, h, WrT[l], WtT[l], b[l], T, N,
                        last=(l == L - 1))
    return h


# speedup vs baseline: 1.5775x; 1.0017x over previous
import functools

import jax
import jax.numpy as jnp
from jax import lax
from jax.experimental import pallas as pl
from jax.experimental.pallas import tpu as pltpu
from jax.experimental.pallas import tpu_sc as plsc

_NC = 2
_NS = 16
_CW = 32
_BB = 128
_NBI = 4
_SB = _BB * _NBI


def _round_up(a, b):
    return (a + b - 1) // b * b


@functools.lru_cache(maxsize=None)
def _make_sc_segsum(N, E_pad, NCH, NKEY, NKEYP, with_counts):
    ZPT = NKEYP // _NS
    EPT = E_pad // _NS
    EPC = E_pad // (_NS * _NC)
    CPC = NCH // _NC
    nsb_data = EPT // _SB
    nsb_cnt = EPC // _SB
    EROWS = E_pad // _BB
    f32 = jnp.float32

    mesh = plsc.VectorSubcoreMesh(core_axis_name="c", subcore_axis_name="s")

    def body(xflat, src8, key2, zeros_h, ones_h, *refs):
        if with_counts:
            s_out, c_out, acc, kbuf, ibuf, dbuf, obuf, gsem, ssem0, ssem1 = refs
        else:
            s_out, acc, kbuf, ibuf, dbuf, obuf, gsem, ssem0, ssem1 = refs
        ssems = (ssem0, ssem1)
        cid = lax.axis_index("c")
        sid = lax.axis_index("s")
        pltpu.sync_copy(ones_h, obuf)

        def zero_acc():
            pltpu.sync_copy(zeros_h, acc.at[pl.ds(sid * ZPT, ZPT)])

        def run_pass(row_base, nsb, chunk):
            gather = chunk is not None

            def stage(slot, sb):
                rk = row_base + sb * _NBI
                pltpu.sync_copy(key2.at[pl.ds(rk, _NBI)], kbuf.at[slot])
                if gather:
                    pltpu.sync_copy(src8.at[pl.ds(chunk * EROWS + rk, _NBI)],
                                    ibuf.at[slot])
                    for j in range(_NBI):
                        pltpu.async_copy(xflat.at[ibuf.at[slot, j]],
                                         dbuf.at[slot, j], gsem)

            def scat_src(slot, j):
                return dbuf.at[slot, j] if gather else obuf

            def wait_scats(slot):
                for j in range(_NBI):
                    pltpu.make_async_copy(scat_src(slot, j),
                                          acc.at[kbuf.at[slot, j]],
                                          ssems[slot]).wait()

            stage(0, 0)

            def outer(o, carry):
                for b in range(2):
                    sb = 2 * o + b
                    ob = 1 - b
                    if gather:
                        for j in range(_NBI):
                            pltpu.make_async_copy(xflat.at[ibuf.at[b, j]],
                                                  dbuf.at[b, j], gsem).wait()
                    for j in range(_NBI):
                        pltpu.async_copy(scat_src(b, j),
                                         acc.at[kbuf.at[b, j]], ssems[b],
                                         add=True)

                    @pl.when(sb > 0)
                    def _free_other():
                        wait_scats(ob)

                    @pl.when(sb + 1 < nsb)
                    def _stage_next():
                        stage(ob, sb + 1)
                return carry

            lax.fori_loop(0, nsb // 2, outer, 0)
            if nsb % 2 == 1:
                b = (nsb - 1) % 2
                if gather:
                    for j in range(_NBI):
                        pltpu.make_async_copy(xflat.at[ibuf.at[b, j]],
                                              dbuf.at[b, j], gsem).wait()
                for j in range(_NBI):
                    pltpu.async_copy(scat_src(b, j), acc.at[kbuf.at[b, j]],
                                     ssems[b], add=True)
                wait_scats(1 - b)
            wait_scats((nsb - 1) % 2)

        for p in range(CPC):
            chunk = cid * CPC + p
            zero_acc()
            plsc.subcore_barrier()
            run_pass(sid * (EPT // _BB), nsb_data, chunk)
            plsc.subcore_barrier()
            pltpu.sync_copy(acc.at[pl.ds(sid * ZPT, ZPT)],
                            s_out.at[cid, pl.ds(sid * ZPT, ZPT),
                                     pl.ds(p * _CW, _CW)])
            plsc.subcore_barrier()

        if with_counts:
            zero_acc()
            plsc.subcore_barrier()
            run_pass(cid * (EPC * _NS // _BB) + sid * (EPC // _BB), nsb_cnt, None)
            plsc.subcore_barrier()
            pltpu.sync_copy(acc.at[pl.ds(sid * ZPT, ZPT)],
                            c_out.at[cid, pl.ds(sid * ZPT, ZPT),
                                     pl.ds(0, _CW)])

    out_type = [jax.ShapeDtypeStruct((_NC, NKEYP, CPC * _CW), f32)]
    if with_counts:
        out_type.append(jax.ShapeDtypeStruct((_NC, NKEYP, CPC * _CW), f32))

    return pl.kernel(
        body,
        out_type=out_type,
        mesh=mesh,
        compiler_params=pltpu.CompilerParams(use_tc_tiling_on_sc=False),
        scratch_types=[
            pltpu.VMEM_SHARED((NKEYP, _CW), f32),
            pltpu.VMEM((2, _NBI, _BB), jnp.int32),
            pltpu.VMEM((2, _NBI, _BB), jnp.int32),
            pltpu.VMEM((2, _NBI, _BB, _CW), f32),
            pltpu.VMEM((_BB, _CW), f32),
            pltpu.SemaphoreType.DMA,
            pltpu.SemaphoreType.DMA,
            pltpu.SemaphoreType.DMA,
        ],
    )


def _tc_combine(s128, c128, x, WrT_l, WtT_l, b_l, T, N, last):
    NC, NKEYP, KW = s128.shape
    D = x.shape[1]
    BN = 1000
    assert N % BN == 0
    NJ = N // BN

    def body(s_ref, c_ref, x_ref, wr_ref, wt_ref, b_ref, o_ref):
        i = pl.program_id(1)

        @pl.when(i == 0)
        def _root():
            o_ref[...] = jnp.dot(x_ref[...], wt_ref[...],
                                 preferred_element_type=jnp.float32) + b_ref[...]

        @pl.when(i > 0)
        def _rel():
            cs = c_ref[...]
            cv = cs[0, :, 0:1] + cs[1, :, 0:1]
            inv = 1.0 / jnp.maximum(cv, 1.0)
            acc = o_ref[...]
            for e in range(NC):
                acc = acc + jnp.dot(s_ref[e] * inv,
                                    wr_ref[0][e * KW:(e + 1) * KW, :],
                                    preferred_element_type=jnp.float32)
            o_ref[...] = acc

        @pl.when(i == T)
        def _act():
            acc = o_ref[...]
            if last:
                m = jnp.max(acc, axis=-1, keepdims=True)
                ex = jnp.exp(acc - m)
                o_ref[...] = acc - m - jnp.log(jnp.sum(ex, axis=-1,
                                                       keepdims=True))
            else:
                o_ref[...] = jnp.maximum(acc, 0.0)

    def ti(i):
        return jnp.maximum(i - 1, 0)

    return pl.pallas_call(
        body,
        grid=(NJ, T + 1),
        in_specs=[
            pl.BlockSpec((NC, BN, KW), lambda j, i: (0, ti(i) * NJ + j, 0)),
            pl.BlockSpec((NC, BN, KW), lambda j, i: (0, ti(i) * NJ + j, 0)),
            pl.BlockSpec((BN, D), lambda j, i: (j, 0)),
            pl.BlockSpec((1, D, D), lambda j, i: (ti(i), 0, 0)),
            pl.BlockSpec((D, D), lambda j, i: (0, 0)),
            pl.BlockSpec((1, D), lambda j, i: (0, 0)),
        ],
        out_specs=pl.BlockSpec((BN, D), lambda j, i: (j, 0)),
        out_shape=jax.ShapeDtypeStruct((N, D), jnp.float32),
    )(s128, c128, x, WrT_l, WtT_l, b_l.reshape(1, D))


def kernel(x_dict, edge_index, edge_type, node_type, local_node_idx,
           W_rel, W_root, b_root):
    N, D = x_dict.shape
    E = edge_index.shape[1]
    L, T = W_rel.shape[0], W_rel.shape[1]
    NCH = D // _CW
    NKEY = T * N
    NKEYP = _round_up(NKEY + 1, _NS * 8)
    E_pad = _round_up(E, _NS * _NC * _SB)

    src = edge_index[0]
    dst = edge_index[1]
    pad = E_pad - E
    key = edge_type * N + dst
    keyp = jnp.concatenate([key, jnp.full((pad,), NKEY, jnp.int32)])
    srcp = jnp.concatenate([src, jnp.zeros((pad,), jnp.int32)])
    key2 = keyp.reshape(E_pad // _BB, _BB)
    src8 = (srcp[None, :] + (jnp.arange(NCH, dtype=jnp.int32) * N)[:, None])
    src8 = src8.reshape(NCH * E_pad // _BB, _BB)
    zeros_h = jnp.zeros((NKEYP // _NS, _CW), jnp.float32)
    ones_h = jnp.ones((_BB, _CW), jnp.float32)

    WrT = W_rel.transpose(0, 1, 3, 2)
    WtT = W_root[:, 0].transpose(0, 2, 1)
    b = b_root[:, 0]

    def chunkify(h):
        return h.reshape(N, NCH, _CW).transpose(1, 0, 2).reshape(NCH * N, _CW)

    h = x_dict
    cnt = None
    for l in range(L):
        sc = _make_sc_segsum(N, E_pad, NCH, NKEY, NKEYP, l == 0)
        outs = sc(chunkify(h), src8, key2, zeros_h, ones_h)
        if l == 0:
            s128, cnt = outs
        else:
            (s128,) = outs
        h = _tc_combine(s128, cnt, h, WrT[l], WtT[l], b[l], T, N,
                        last=(l == L - 1))
    return h


# speedup vs baseline: 1.7823x; 1.1298x over previous
import functools

import jax
import jax.numpy as jnp
from jax import lax
from jax.experimental import pallas as pl
from jax.experimental.pallas import tpu as pltpu
from jax.experimental.pallas import tpu_sc as plsc

_NC = 2
_NS = 16
_CW = 32
_BB = 128
_NBI = 4
_SB = _BB * _NBI


def _round_up(a, b):
    return (a + b - 1) // b * b


@functools.lru_cache(maxsize=None)
def _make_sc_segsum(N, E_pad, NCH, NKEY, NKEYP, with_counts):
    ZPT = NKEYP // _NS
    EPT = E_pad // _NS
    EPC = E_pad // (_NS * _NC)
    CPC = NCH // _NC
    nsb_data = EPT // _SB
    nsb_cnt = EPC // _SB
    EROWS = E_pad // _BB
    f32 = jnp.float32

    mesh = plsc.VectorSubcoreMesh(core_axis_name="c", subcore_axis_name="s")

    def body(xflat, src8, key2, zeros_h, ones_h, *refs):
        if with_counts:
            (s_out, c_out, acc, kbuf, ibuf, dbuf, obuf,
             gsem, ssem0, ssem1, lsem) = refs
        else:
            (s_out, acc, kbuf, ibuf, dbuf, obuf,
             gsem, ssem0, ssem1, lsem) = refs
        ssems = (ssem0, ssem1)
        cid = lax.axis_index("c")
        sid = lax.axis_index("s")
        pltpu.sync_copy(ones_h, obuf)

        def zero_acc():
            pltpu.sync_copy(zeros_h, acc.at[pl.ds(sid * ZPT, ZPT)])

        def run_pass(row_base, nsb, chunk):
            gather = chunk is not None

            def fire_loads(slot, sb):
                rk = row_base + sb * _NBI
                pltpu.async_copy(key2.at[pl.ds(rk, _NBI)], kbuf.at[slot], lsem)
                if gather:
                    pltpu.async_copy(src8.at[pl.ds(chunk * EROWS + rk, _NBI)],
                                     ibuf.at[slot], lsem)

            def wait_loads(slot):
                pltpu.make_async_copy(key2.at[pl.ds(row_base, _NBI)],
                                      kbuf.at[slot], lsem).wait()
                if gather:
                    pltpu.make_async_copy(src8.at[pl.ds(row_base, _NBI)],
                                          ibuf.at[slot], lsem).wait()

            def scat_src(slot, j):
                return dbuf.at[slot, j] if gather else obuf

            def wait_scats(slot):
                for j in range(_NBI):
                    pltpu.make_async_copy(scat_src(slot, j),
                                          acc.at[kbuf.at[slot, j]],
                                          ssems[slot]).wait()

            def onestep(b, sb):
                ob = 1 - b
                wait_loads(b)
                if gather:
                    for j in range(_NBI):
                        pltpu.async_copy(xflat.at[ibuf.at[b, j]],
                                         dbuf.at[b, j], gsem)

                @pl.when(sb > 0)
                def _free_other():
                    wait_scats(ob)

                @pl.when(sb + 1 < nsb)
                def _load_next():
                    fire_loads(ob, sb + 1)

                if gather:
                    for j in range(_NBI):
                        pltpu.make_async_copy(xflat.at[ibuf.at[b, j]],
                                              dbuf.at[b, j], gsem).wait()
                for j in range(_NBI):
                    pltpu.async_copy(scat_src(b, j), acc.at[kbuf.at[b, j]],
                                     ssems[b], add=True)

            fire_loads(0, 0)

            def outer(o, carry):
                for b in range(2):
                    onestep(b, 2 * o + b)
                return carry

            lax.fori_loop(0, nsb // 2, outer, 0)
            if nsb % 2 == 1:
                onestep((nsb - 1) % 2, nsb - 1)
            wait_scats((nsb - 1) % 2)

        for p in range(CPC):
            chunk = cid * CPC + p
            zero_acc()
            plsc.subcore_barrier()
            run_pass(sid * (EPT // _BB), nsb_data, chunk)
            plsc.subcore_barrier()
            pltpu.sync_copy(acc.at[pl.ds(sid * ZPT, ZPT)],
                            s_out.at[cid, pl.ds(sid * ZPT, ZPT),
                                     pl.ds(p * _CW, _CW)])
            plsc.subcore_barrier()

        if with_counts:
            zero_acc()
            plsc.subcore_barrier()
            run_pass(cid * (EPC * _NS // _BB) + sid * (EPC // _BB), nsb_cnt, None)
            plsc.subcore_barrier()
            pltpu.sync_copy(acc.at[pl.ds(sid * ZPT, ZPT)],
                            c_out.at[cid, pl.ds(sid * ZPT, ZPT),
                                     pl.ds(0, _CW)])

    out_type = [jax.ShapeDtypeStruct((_NC, NKEYP, CPC * _CW), f32)]
    if with_counts:
        out_type.append(jax.ShapeDtypeStruct((_NC, NKEYP, CPC * _CW), f32))

    return pl.kernel(
        body,
        out_type=out_type,
        mesh=mesh,
        compiler_params=pltpu.CompilerParams(use_tc_tiling_on_sc=False),
        scratch_types=[
            pltpu.VMEM_SHARED((NKEYP, _CW), f32),
            pltpu.VMEM((2, _NBI, _BB), jnp.int32),
            pltpu.VMEM((2, _NBI, _BB), jnp.int32),
            pltpu.VMEM((2, _NBI, _BB, _CW), f32),
            pltpu.VMEM((_BB, _CW), f32),
            pltpu.SemaphoreType.DMA,
            pltpu.SemaphoreType.DMA,
            pltpu.SemaphoreType.DMA,
            pltpu.SemaphoreType.DMA,
        ],
    )


def _tc_combine(s128, c128, x, WrT_l, WtT_l, b_l, T, N, last):
    NC, NKEYP, KW = s128.shape
    D = x.shape[1]
    BN = 1000
    assert N % BN == 0
    NJ = N // BN

    def body(s_ref, c_ref, x_ref, wr_ref, wt_ref, b_ref, o_ref):
        i = pl.program_id(1)

        @pl.when(i == 0)
        def _root():
            o_ref[...] = jnp.dot(x_ref[...], wt_ref[...],
                                 preferred_element_type=jnp.float32) + b_ref[...]

        @pl.when(i > 0)
        def _rel():
            cs = c_ref[...]
            cv = cs[0, :, 0:1] + cs[1, :, 0:1]
            inv = 1.0 / jnp.maximum(cv, 1.0)
            acc = o_ref[...]
            for e in range(NC):
                acc = acc + jnp.dot(s_ref[e] * inv,
                                    wr_ref[0][e * KW:(e + 1) * KW, :],
                                    preferred_element_type=jnp.float32)
            o_ref[...] = acc

        @pl.when(i == T)
        def _act():
            acc = o_ref[...]
            if last:
                m = jnp.max(acc, axis=-1, keepdims=True)
                ex = jnp.exp(acc - m)
                o_ref[...] = acc - m - jnp.log(jnp.sum(ex, axis=-1,
                                                       keepdims=True))
            else:
                o_ref[...] = jnp.maximum(acc, 0.0)

    def ti(i):
        return jnp.maximum(i - 1, 0)

    return pl.pallas_call(
        body,
        grid=(NJ, T + 1),
        in_specs=[
            pl.BlockSpec((NC, BN, KW), lambda j, i: (0, ti(i) * NJ + j, 0)),
            pl.BlockSpec((NC, BN, KW), lambda j, i: (0, ti(i) * NJ + j, 0)),
            pl.BlockSpec((BN, D), lambda j, i: (j, 0)),
            pl.BlockSpec((1, D, D), lambda j, i: (ti(i), 0, 0)),
            pl.BlockSpec((D, D), lambda j, i: (0, 0)),
            pl.BlockSpec((1, D), lambda j, i: (0, 0)),
        ],
        out_specs=pl.BlockSpec((BN, D), lambda j, i: (j, 0)),
        out_shape=jax.ShapeDtypeStruct((N, D), jnp.float32),
    )(s128, c128, x, WrT_l, WtT_l, b_l.reshape(1, D))


def kernel(x_dict, edge_index, edge_type, node_type, local_node_idx,
           W_rel, W_root, b_root):
    N, D = x_dict.shape
    E = edge_index.shape[1]
    L, T = W_rel.shape[0], W_rel.shape[1]
    NCH = D // _CW
    NKEY = T * N
    NKEYP = _round_up(NKEY + 1, _NS * 8)
    E_pad = _round_up(E, _NS * _NC * _SB)

    src = edge_index[0]
    dst = edge_index[1]
    pad = E_pad - E
    key = edge_type * N + dst
    keyp = jnp.concatenate([key, jnp.full((pad,), NKEY, jnp.int32)])
    srcp = jnp.concatenate([src, jnp.zeros((pad,), jnp.int32)])
    key2 = keyp.reshape(E_pad // _BB, _BB)
    src8 = (srcp[None, :] + (jnp.arange(NCH, dtype=jnp.int32) * N)[:, None])
    src8 = src8.reshape(NCH * E_pad // _BB, _BB)
    zeros_h = jnp.zeros((NKEYP // _NS, _CW), jnp.float32)
    ones_h = jnp.ones((_BB, _CW), jnp.float32)

    WrT = W_rel.transpose(0, 1, 3, 2)
    WtT = W_root[:, 0].transpose(0, 2, 1)
    b = b_root[:, 0]

    def chunkify(h):
        return h.reshape(N, NCH, _CW).transpose(1, 0, 2).reshape(NCH * N, _CW)

    h = x_dict
    cnt = None
    for l in range(L):
        sc = _make_sc_segsum(N, E_pad, NCH, NKEY, NKEYP, l == 0)
        outs = sc(chunkify(h), src8, key2, zeros_h, ones_h)
        if l == 0:
            s128, cnt = outs
        else:
            (s128,) = outs
        h = _tc_combine(s128, cnt, h, WrT[l], WtT[l], b[l], T, N,
                        last=(l == L - 1))
    return h


# speedup vs baseline: 1.8075x; 1.0142x over previous
import functools

import jax
import jax.numpy as jnp
from jax import lax
from jax.experimental import pallas as pl
from jax.experimental.pallas import tpu as pltpu
from jax.experimental.pallas import tpu_sc as plsc

_NC = 2
_NS = 16
_CW = 32
_BB = 128
_NBI = 5
_SB = _BB * _NBI


def _round_up(a, b):
    return (a + b - 1) // b * b


@functools.lru_cache(maxsize=None)
def _make_sc_segsum(N, E_pad, NCH, NKEY, NKEYP, with_counts):
    ZPT = NKEYP // _NS
    EPT = E_pad // _NS
    EPC = E_pad // (_NS * _NC)
    CPC = NCH // _NC
    nsb_data = EPT // _SB
    nsb_cnt = EPC // _SB
    EROWS = E_pad // _BB
    f32 = jnp.float32

    mesh = plsc.VectorSubcoreMesh(core_axis_name="c", subcore_axis_name="s")

    def body(xflat, src8, key2, zeros_h, ones_h, *refs):
        if with_counts:
            (s_out, c_out, acc, kbuf, ibuf, dbuf, obuf,
             gsem, ssem0, ssem1, lsem) = refs
        else:
            (s_out, acc, kbuf, ibuf, dbuf, obuf,
             gsem, ssem0, ssem1, lsem) = refs
        ssems = (ssem0, ssem1)
        cid = lax.axis_index("c")
        sid = lax.axis_index("s")
        pltpu.sync_copy(ones_h, obuf)

        def zero_acc():
            pltpu.sync_copy(zeros_h, acc.at[pl.ds(sid * ZPT, ZPT)])

        def run_pass(row_base, nsb, chunk):
            gather = chunk is not None

            def fire_loads(slot, sb):
                rk = row_base + sb * _NBI
                pltpu.async_copy(key2.at[pl.ds(rk, _NBI)], kbuf.at[slot], lsem)
                if gather:
                    pltpu.async_copy(src8.at[pl.ds(chunk * EROWS + rk, _NBI)],
                                     ibuf.at[slot], lsem)

            def wait_loads(slot):
                pltpu.make_async_copy(key2.at[pl.ds(row_base, _NBI)],
                                      kbuf.at[slot], lsem).wait()
                if gather:
                    pltpu.make_async_copy(src8.at[pl.ds(row_base, _NBI)],
                                          ibuf.at[slot], lsem).wait()

            def scat_src(slot, j):
                return dbuf.at[slot, j] if gather else obuf

            def wait_scats(slot):
                for j in range(_NBI):
                    pltpu.make_async_copy(scat_src(slot, j),
                                          acc.at[kbuf.at[slot, j]],
                                          ssems[slot]).wait()

            def onestep(b, sb):
                ob = 1 - b
                wait_loads(b)
                if gather:
                    for j in range(_NBI):
                        pltpu.async_copy(xflat.at[ibuf.at[b, j]],
                                         dbuf.at[b, j], gsem)

                @pl.when(sb > 0)
                def _free_other():
                    wait_scats(ob)

                @pl.when(sb + 1 < nsb)
                def _load_next():
                    fire_loads(ob, sb + 1)

                if gather:
                    for j in range(_NBI):
                        pltpu.make_async_copy(xflat.at[ibuf.at[b, j]],
                                              dbuf.at[b, j], gsem).wait()
                for j in range(_NBI):
                    pltpu.async_copy(scat_src(b, j), acc.at[kbuf.at[b, j]],
                                     ssems[b], add=True)

            fire_loads(0, 0)

            def outer(o, carry):
                for b in range(2):
                    onestep(b, 2 * o + b)
                return carry

            lax.fori_loop(0, nsb // 2, outer, 0)
            if nsb % 2 == 1:
                onestep((nsb - 1) % 2, nsb - 1)
            wait_scats((nsb - 1) % 2)

        for p in range(CPC):
            chunk = cid * CPC + p
            zero_acc()
            plsc.subcore_barrier()
            run_pass(sid * (EPT // _BB), nsb_data, chunk)
            plsc.subcore_barrier()
            pltpu.sync_copy(acc.at[pl.ds(sid * ZPT, ZPT)],
                            s_out.at[cid, pl.ds(sid * ZPT, ZPT),
                                     pl.ds(p * _CW, _CW)])
            plsc.subcore_barrier()

        if with_counts:
            zero_acc()
            plsc.subcore_barrier()
            run_pass(cid * (EPC * _NS // _BB) + sid * (EPC // _BB), nsb_cnt, None)
            plsc.subcore_barrier()
            pltpu.sync_copy(acc.at[pl.ds(sid * ZPT, ZPT)],
                            c_out.at[cid, pl.ds(sid * ZPT, ZPT),
                                     pl.ds(0, _CW)])

    out_type = [jax.ShapeDtypeStruct((_NC, NKEYP, CPC * _CW), f32)]
    if with_counts:
        out_type.append(jax.ShapeDtypeStruct((_NC, NKEYP, CPC * _CW), f32))

    return pl.kernel(
        body,
        out_type=out_type,
        mesh=mesh,
        compiler_params=pltpu.CompilerParams(use_tc_tiling_on_sc=False),
        scratch_types=[
            pltpu.VMEM_SHARED((NKEYP, _CW), f32),
            pltpu.VMEM((2, _NBI, _BB), jnp.int32),
            pltpu.VMEM((2, _NBI, _BB), jnp.int32),
            pltpu.VMEM((2, _NBI, _BB, _CW), f32),
            pltpu.VMEM((_BB, _CW), f32),
            pltpu.SemaphoreType.DMA,
            pltpu.SemaphoreType.DMA,
            pltpu.SemaphoreType.DMA,
            pltpu.SemaphoreType.DMA,
        ],
    )


def _tc_combine(s128, c128, x, WrT_l, WtT_l, b_l, T, N, last):
    NC, NKEYP, KW = s128.shape
    D = x.shape[1]
    BN = 1000
    assert N % BN == 0
    NJ = N // BN

    def body(s_ref, c_ref, x_ref, wr_ref, wt_ref, b_ref, o_ref):
        i = pl.program_id(1)

        @pl.when(i == 0)
        def _root():
            o_ref[...] = jnp.dot(x_ref[...], wt_ref[...],
                                 preferred_element_type=jnp.float32) + b_ref[...]

        @pl.when(i > 0)
        def _rel():
            cs = c_ref[...]
            cv = cs[0, :, 0:1] + cs[1, :, 0:1]
            inv = 1.0 / jnp.maximum(cv, 1.0)
            acc = o_ref[...]
            for e in range(NC):
                acc = acc + jnp.dot(s_ref[e] * inv,
                                    wr_ref[0][e * KW:(e + 1) * KW, :],
                                    preferred_element_type=jnp.float32)
            o_ref[...] = acc

        @pl.when(i == T)
        def _act():
            acc = o_ref[...]
            if last:
                m = jnp.max(acc, axis=-1, keepdims=True)
                ex = jnp.exp(acc - m)
                o_ref[...] = acc - m - jnp.log(jnp.sum(ex, axis=-1,
                                                       keepdims=True))
            else:
                o_ref[...] = jnp.maximum(acc, 0.0)

    def ti(i):
        return jnp.maximum(i - 1, 0)

    return pl.pallas_call(
        body,
        grid=(NJ, T + 1),
        in_specs=[
            pl.BlockSpec((NC, BN, KW), lambda j, i: (0, ti(i) * NJ + j, 0)),
            pl.BlockSpec((NC, BN, KW), lambda j, i: (0, ti(i) * NJ + j, 0)),
            pl.BlockSpec((BN, D), lambda j, i: (j, 0)),
            pl.BlockSpec((1, D, D), lambda j, i: (ti(i), 0, 0)),
            pl.BlockSpec((D, D), lambda j, i: (0, 0)),
            pl.BlockSpec((1, D), lambda j, i: (0, 0)),
        ],
        out_specs=pl.BlockSpec((BN, D), lambda j, i: (j, 0)),
        out_shape=jax.ShapeDtypeStruct((N, D), jnp.float32),
    )(s128, c128, x, WrT_l, WtT_l, b_l.reshape(1, D))


def kernel(x_dict, edge_index, edge_type, node_type, local_node_idx,
           W_rel, W_root, b_root):
    N, D = x_dict.shape
    E = edge_index.shape[1]
    L, T = W_rel.shape[0], W_rel.shape[1]
    NCH = D // _CW
    NKEY = T * N
    NKEYP = _round_up(NKEY + 1, _NS * 8)
    E_pad = _round_up(E, _NS * _NC * _SB)

    src = edge_index[0]
    dst = edge_index[1]
    pad = E_pad - E
    key = edge_type * N + dst
    keyp = jnp.concatenate([key, jnp.full((pad,), NKEY, jnp.int32)])
    srcp = jnp.concatenate([src, jnp.zeros((pad,), jnp.int32)])
    key2 = keyp.reshape(E_pad // _BB, _BB)
    src8 = (srcp[None, :] + (jnp.arange(NCH, dtype=jnp.int32) * N)[:, None])
    src8 = src8.reshape(NCH * E_pad // _BB, _BB)
    zeros_h = jnp.zeros((NKEYP // _NS, _CW), jnp.float32)
    ones_h = jnp.ones((_BB, _CW), jnp.float32)

    WrT = W_rel.transpose(0, 1, 3, 2)
    WtT = W_root[:, 0].transpose(0, 2, 1)
    b = b_root[:, 0]

    def chunkify(h):
        return h.reshape(N, NCH, _CW).transpose(1, 0, 2).reshape(NCH * N, _CW)

    h = x_dict
    cnt = None
    for l in range(L):
        sc = _make_sc_segsum(N, E_pad, NCH, NKEY, NKEYP, l == 0)
        outs = sc(chunkify(h), src8, key2, zeros_h, ones_h)
        if l == 0:
            s128, cnt = outs
        else:
            (s128,) = outs
        h = _tc_combine(s128, cnt, h, WrT[l], WtT[l], b[l], T, N,
                        last=(l == L - 1))
    return h


# speedup vs baseline: 1.8812x; 1.0408x over previous
import functools

import jax
import jax.numpy as jnp
from jax import lax
from jax.experimental import pallas as pl
from jax.experimental.pallas import tpu as pltpu
from jax.experimental.pallas import tpu_sc as plsc

_NC = 2
_NS = 16
_CW = 32
_BB = 128
_NBI = 5
_SB = _BB * _NBI


def _round_up(a, b):
    return (a + b - 1) // b * b


@functools.lru_cache(maxsize=None)
def _make_sc_segsum(N, E_pad, NCH, NKEY, NKEYP, with_counts):
    ZPT = NKEYP // _NS
    EPT = E_pad // _NS
    EPC = E_pad // (_NS * _NC)
    CPC = NCH // _NC
    nsb_data = EPT // _SB
    nsb_cnt = EPC // _SB
    EROWS = E_pad // _BB
    f32 = jnp.float32

    mesh = plsc.VectorSubcoreMesh(core_axis_name="c", subcore_axis_name="s")

    def body(xflat, src2, key2, zeros_h, ones_h, *refs):
        if with_counts:
            (s_out, c_out, acc, kbuf, ibuf, dbuf, obuf,
             gsem, ssem0, ssem1, lsem) = refs
        else:
            (s_out, acc, kbuf, ibuf, dbuf, obuf,
             gsem, ssem0, ssem1, lsem) = refs
        ssems = (ssem0, ssem1)
        cid = lax.axis_index("c")
        sid = lax.axis_index("s")
        pltpu.sync_copy(ones_h, obuf)

        def zero_acc():
            pltpu.sync_copy(zeros_h, acc.at[pl.ds(sid * ZPT, ZPT)])

        def run_pass(row_base, nsb, chunk):
            gather = chunk is not None

            def fire_loads(slot, sb):
                rk = row_base + sb * _NBI
                pltpu.async_copy(key2.at[pl.ds(rk, _NBI)], kbuf.at[slot], lsem)
                if gather:
                    pltpu.async_copy(src2.at[pl.ds(rk, _NBI)],
                                     ibuf.at[slot], lsem)

            def wait_loads(slot):
                pltpu.make_async_copy(key2.at[pl.ds(row_base, _NBI)],
                                      kbuf.at[slot], lsem).wait()
                if gather:
                    pltpu.make_async_copy(src2.at[pl.ds(row_base, _NBI)],
                                          ibuf.at[slot], lsem).wait()
                    off = chunk * N
                    for r in range(_NBI):
                        for v in range(_BB // 16):
                            sl = ibuf[slot, r, pl.ds(v * 16, 16)]
                            ibuf[slot, r, pl.ds(v * 16, 16)] = sl + off

            def scat_src(slot, j):
                return dbuf.at[slot, j] if gather else obuf

            def wait_scats(slot):
                for j in range(_NBI):
                    pltpu.make_async_copy(scat_src(slot, j),
                                          acc.at[kbuf.at[slot, j]],
                                          ssems[slot]).wait()

            def onestep(b, sb):
                ob = 1 - b
                wait_loads(b)
                if gather:
                    for j in range(_NBI):
                        pltpu.async_copy(xflat.at[ibuf.at[b, j]],
                                         dbuf.at[b, j], gsem)

                @pl.when(sb > 0)
                def _free_other():
                    wait_scats(ob)

                @pl.when(sb + 1 < nsb)
                def _load_next():
                    fire_loads(ob, sb + 1)

                if gather:
                    for j in range(_NBI):
                        pltpu.make_async_copy(xflat.at[ibuf.at[b, j]],
                                              dbuf.at[b, j], gsem).wait()
                for j in range(_NBI):
                    pltpu.async_copy(scat_src(b, j), acc.at[kbuf.at[b, j]],
                                     ssems[b], add=True)

            fire_loads(0, 0)

            def outer(o, carry):
                for b in range(2):
                    onestep(b, 2 * o + b)
                return carry

            lax.fori_loop(0, nsb // 2, outer, 0)
            if nsb % 2 == 1:
                onestep((nsb - 1) % 2, nsb - 1)
            wait_scats((nsb - 1) % 2)

        zero_acc()
        plsc.subcore_barrier()
        for p in range(CPC):
            chunk = cid * CPC + p
            run_pass(sid * (EPT // _BB), nsb_data, chunk)
            plsc.subcore_barrier()
            pltpu.sync_copy(acc.at[pl.ds(sid * ZPT, ZPT)],
                            s_out.at[cid, pl.ds(sid * ZPT, ZPT),
                                     pl.ds(p * _CW, _CW)])
            if p + 1 < CPC or with_counts:
                zero_acc()
            plsc.subcore_barrier()

        if with_counts:
            run_pass(cid * (EPC * _NS // _BB) + sid * (EPC // _BB), nsb_cnt, None)
            plsc.subcore_barrier()
            pltpu.sync_copy(acc.at[pl.ds(sid * ZPT, ZPT)],
                            c_out.at[cid, pl.ds(sid * ZPT, ZPT),
                                     pl.ds(0, _CW)])

    out_type = [jax.ShapeDtypeStruct((_NC, NKEYP, CPC * _CW), f32)]
    if with_counts:
        out_type.append(jax.ShapeDtypeStruct((_NC, NKEYP, CPC * _CW), f32))

    return pl.kernel(
        body,
        out_type=out_type,
        mesh=mesh,
        compiler_params=pltpu.CompilerParams(use_tc_tiling_on_sc=False),
        scratch_types=[
            pltpu.VMEM_SHARED((NKEYP, _CW), f32),
            pltpu.VMEM((2, _NBI, _BB), jnp.int32),
            pltpu.VMEM((2, _NBI, _BB), jnp.int32),
            pltpu.VMEM((2, _NBI, _BB, _CW), f32),
            pltpu.VMEM((_BB, _CW), f32),
            pltpu.SemaphoreType.DMA,
            pltpu.SemaphoreType.DMA,
            pltpu.SemaphoreType.DMA,
            pltpu.SemaphoreType.DMA,
        ],
    )


def _tc_combine(s128, c128, x, WrT_l, WtT_l, b_l, T, N, last):
    NC, NKEYP, KW = s128.shape
    D = x.shape[1]
    BN = 1000
    assert N % BN == 0
    NJ = N // BN

    def body(s_ref, c_ref, x_ref, wr_ref, wt_ref, b_ref, o_ref):
        i = pl.program_id(1)

        @pl.when(i == 0)
        def _root():
            o_ref[...] = jnp.dot(x_ref[...], wt_ref[...],
                                 preferred_element_type=jnp.float32) + b_ref[...]

        @pl.when(i > 0)
        def _rel():
            cs = c_ref[...]
            cv = cs[0, :, 0:1] + cs[1, :, 0:1]
            inv = 1.0 / jnp.maximum(cv, 1.0)
            acc = o_ref[...]
            for e in range(NC):
                acc = acc + jnp.dot(s_ref[e] * inv,
                                    wr_ref[0][e * KW:(e + 1) * KW, :],
                                    preferred_element_type=jnp.float32)
            o_ref[...] = acc

        @pl.when(i == T)
        def _act():
            acc = o_ref[...]
            if last:
                m = jnp.max(acc, axis=-1, keepdims=True)
                ex = jnp.exp(acc - m)
                o_ref[...] = acc - m - jnp.log(jnp.sum(ex, axis=-1,
                                                       keepdims=True))
            else:
                o_ref[...] = jnp.maximum(acc, 0.0)

    def ti(i):
        return jnp.maximum(i - 1, 0)

    return pl.pallas_call(
        body,
        grid=(NJ, T + 1),
        in_specs=[
            pl.BlockSpec((NC, BN, KW), lambda j, i: (0, ti(i) * NJ + j, 0)),
            pl.BlockSpec((NC, BN, KW), lambda j, i: (0, ti(i) * NJ + j, 0)),
            pl.BlockSpec((BN, D), lambda j, i: (j, 0)),
            pl.BlockSpec((1, D, D), lambda j, i: (ti(i), 0, 0)),
            pl.BlockSpec((D, D), lambda j, i: (0, 0)),
            pl.BlockSpec((1, D), lambda j, i: (0, 0)),
        ],
        out_specs=pl.BlockSpec((BN, D), lambda j, i: (j, 0)),
        out_shape=jax.ShapeDtypeStruct((N, D), jnp.float32),
    )(s128, c128, x, WrT_l, WtT_l, b_l.reshape(1, D))


def kernel(x_dict, edge_index, edge_type, node_type, local_node_idx,
           W_rel, W_root, b_root):
    N, D = x_dict.shape
    E = edge_index.shape[1]
    L, T = W_rel.shape[0], W_rel.shape[1]
    NCH = D // _CW
    NKEY = T * N
    NKEYP = _round_up(NKEY + 1, _NS * 8)
    E_pad = _round_up(E, _NS * _NC * _SB)

    src = edge_index[0]
    dst = edge_index[1]
    pad = E_pad - E
    key = edge_type * N + dst
    keyp = jnp.concatenate([key, jnp.full((pad,), NKEY, jnp.int32)])
    srcp = jnp.concatenate([src, jnp.zeros((pad,), jnp.int32)])
    key2 = keyp.reshape(E_pad // _BB, _BB)
    src2 = srcp.reshape(E_pad // _BB, _BB)
    zeros_h = jnp.zeros((NKEYP // _NS, _CW), jnp.float32)
    ones_h = jnp.ones((_BB, _CW), jnp.float32)

    WrT = W_rel.transpose(0, 1, 3, 2)
    WtT = W_root[:, 0].transpose(0, 2, 1)
    b = b_root[:, 0]

    def chunkify(h):
        return h.reshape(N, NCH, _CW).transpose(1, 0, 2).reshape(NCH * N, _CW)

    h = x_dict
    cnt = None
    for l in range(L):
        sc = _make_sc_segsum(N, E_pad, NCH, NKEY, NKEYP, l == 0)
        outs = sc(chunkify(h), src2, key2, zeros_h, ones_h)
        if l == 0:
            s128, cnt = outs
        else:
            (s128,) = outs
        h = _tc_combine(s128, cnt, h, WrT[l], WtT[l], b[l], T, N,
                        last=(l == L - 1))
    return h
